# Initial kernel scaffold; baseline (speedup 1.0000x reference)
import jax
import jax.numpy as jnp
from jax.experimental import pallas as pl


def kernel(inputs, emb, k_f, rk_f, b_f, k_b, rk_b, b_b, W1, b1, W2, b2):
    raise NotImplementedError("write your pallas kernel here")


# speedup vs baseline: 1.4244x; 1.4244x over previous
import functools

import jax
import jax.numpy as jnp
from jax import lax
from jax.experimental import pallas as pl
from jax.experimental.pallas import tpu as pltpu
from jax.experimental.pallas import tpu_sc as plsc

H = 64
_PREC = jax.lax.Precision.HIGHEST


def _sc_embedding_gather(table, flat_idx):
    n = flat_idx.shape[0]
    d = table.shape[1]
    info = plsc.get_sparse_core_info()
    nw = info.num_cores * info.num_subcores
    sub = 128
    subs_per_chunk = 8
    chunk = sub * subs_per_chunk
    per_w = n // nw
    n_chunks = per_w // chunk
    idx2d = flat_idx.reshape(n // sub, sub)
    mesh = plsc.VectorSubcoreMesh(core_axis_name="c", subcore_axis_name="s")

    @functools.partial(
        pl.kernel,
        mesh=mesh,
        compiler_params=pltpu.CompilerParams(use_tc_tiling_on_sc=False),
        out_type=jax.ShapeDtypeStruct((n, d), jnp.float32),
        scratch_types=[
            pltpu.VMEM((subs_per_chunk, sub), jnp.int32),
            pltpu.VMEM((chunk, d), jnp.float32),
            pltpu.SemaphoreType.DMA,
        ],
    )
    def gather_kernel(table_hbm, idx_hbm, out_hbm, idx_v, rows_v, sem):
        wid = lax.axis_index("s") * info.num_cores + lax.axis_index("c")
        row_base = wid * per_w
        irow_base = row_base // sub

        def body(i, carry):
            row0 = pl.multiple_of(row_base + i * chunk, chunk)
            irow0 = pl.multiple_of(irow_base + i * subs_per_chunk,
                                   subs_per_chunk)
            pltpu.sync_copy(idx_hbm.at[pl.ds(irow0, subs_per_chunk)], idx_v)
            copies = [
                pltpu.async_copy(
                    table_hbm.at[idx_v.at[j]],
                    rows_v.at[pl.ds(j * sub, sub)],
                    sem,
                )
                for j in range(subs_per_chunk)
            ]
            for c in copies:
                c.wait()
            pltpu.sync_copy(rows_v, out_hbm.at[pl.ds(row0, chunk)])
            return carry

        lax.fori_loop(0, n_chunks, body, 0)

    return gather_kernel(table, idx2d)


def _gru_scan(x_lbd, ids_lb1, kern, rk, bias, reverse):
    L, B, D = x_lbd.shape

    def body(x_ref, ids_ref, k_ref, rk_ref, b_ref, out_ref, h_ref):
        i = pl.program_id(0)
        t = (L - 1 - i) if reverse else i

        @pl.when(i == 0)
        def _init():
            h_ref[...] = jnp.zeros_like(h_ref)

        h = h_ref[...]
        x_t = x_ref[0]
        gx = (
            jnp.dot(x_t, k_ref[...], preferred_element_type=jnp.float32,
                    precision=_PREC)
            + b_ref[0:1, :]
        )
        gh = (
            jnp.dot(h, rk_ref[...], preferred_element_type=jnp.float32,
                    precision=_PREC)
            + b_ref[1:2, :]
        )
        xz = gx[:, :H]
        xr = gx[:, H:2 * H]
        xh = gx[:, 2 * H:]
        hz = gh[:, :H]
        hr = gh[:, H:2 * H]
        hh = gh[:, 2 * H:]
        z = jax.nn.sigmoid(xz + hz)
        r = jax.nn.sigmoid(xr + hr)
        c = jnp.tanh(xh + r * hh)
        h_new = z * h + (1.0 - z) * c
        m = ids_ref[0] != 0
        h_new = jnp.where(m, h_new, h)
        h_ref[...] = h_new
        out_ref[:, pl.ds(t % 8, 1), :] = h_new[:, None, :]

    if reverse:
        in_idx = lambda i: (L - 1 - i, 0, 0)
        out_idx = lambda i: (0, (L - 1 - i) // 8, 0)
    else:
        in_idx = lambda i: (i, 0, 0)
        out_idx = lambda i: (0, i // 8, 0)

    return pl.pallas_call(
        body,
        grid=(L,),
        in_specs=[
            pl.BlockSpec((1, B, D), in_idx),
            pl.BlockSpec((1, B, 1), in_idx),
            pl.BlockSpec((D, 3 * H), lambda t: (0, 0)),
            pl.BlockSpec((H, 3 * H), lambda t: (0, 0)),
            pl.BlockSpec((2, 3 * H), lambda t: (0, 0)),
        ],
        out_specs=pl.BlockSpec((B, 8, H), out_idx),
        out_shape=jax.ShapeDtypeStruct((B, L, H), jnp.float32),
        scratch_shapes=[pltpu.VMEM((B, H), jnp.float32)],
    )(x_lbd, ids_lb1, kern, rk, bias)


def _heads(fwd, bwd, wt, bb):
    B, L, _ = fwd.shape
    Bb = 64

    def body(f_ref, b_ref, w_ref, bias_ref, xg_ref, x1_ref, x2_ref):
        xg = jnp.concatenate([f_ref[...], b_ref[...]], axis=-1)
        xg_ref[...] = xg
        w = w_ref[...]
        w1 = w[0:1, :].reshape(1, 1, 2 * H)
        w2 = w[1:2, :].reshape(1, 1, 2 * H)
        s1 = jnp.sum(xg * w1, axis=-1) + bias_ref[0, 0]
        s2 = jnp.sum(xg * w2, axis=-1) + bias_ref[0, 1]
        x1_ref[...] = jax.nn.sigmoid(s1)
        x2_ref[...] = jax.nn.sigmoid(s2)

    return pl.pallas_call(
        body,
        grid=(B // Bb,),
        in_specs=[
            pl.BlockSpec((Bb, L, H), lambda i: (i, 0, 0)),
            pl.BlockSpec((Bb, L, H), lambda i: (i, 0, 0)),
            pl.BlockSpec((2, 2 * H), lambda i: (0, 0)),
            pl.BlockSpec((1, 2), lambda i: (0, 0)),
        ],
        out_specs=[
            pl.BlockSpec((Bb, L, 2 * H), lambda i: (i, 0, 0)),
            pl.BlockSpec((Bb, L), lambda i: (i, 0)),
            pl.BlockSpec((Bb, L), lambda i: (i, 0)),
        ],
        out_shape=[
            jax.ShapeDtypeStruct((B, L, 2 * H), jnp.float32),
            jax.ShapeDtypeStruct((B, L), jnp.float32),
            jax.ShapeDtypeStruct((B, L), jnp.float32),
        ],
    )(fwd, bwd, wt, bb)


def kernel(inputs, emb, k_f, rk_f, b_f, k_b, rk_b, b_b, W1, b1, W2, b2):
    B, L = inputs.shape
    ids = inputs.astype(jnp.int32)
    ids_t = ids.T
    x_flat = _sc_embedding_gather(emb, ids_t.reshape(-1))
    x_lbd = x_flat.reshape(L, B, H)
    ids_lb1 = ids_t.reshape(L, B, 1)
    fwd = _gru_scan(x_lbd, ids_lb1, k_f, rk_f, b_f, reverse=False)
    bwd = _gru_scan(x_lbd, ids_lb1, k_b, rk_b, b_b, reverse=True)
    wt = jnp.concatenate([W1, W2], axis=1).T
    bb = jnp.concatenate([b1, b2]).reshape(1, 2)
    xgru, x1, x2 = _heads(fwd, bwd, wt, bb)
    return (x1[..., None], x2[..., None], xgru)


# speedup vs baseline: 2.1052x; 1.4780x over previous
import functools

import jax
import jax.numpy as jnp
from jax import lax
from jax.experimental import pallas as pl
from jax.experimental.pallas import tpu as pltpu
from jax.experimental.pallas import tpu_sc as plsc

H = 64
_PREC = jax.lax.Precision.HIGHEST


def _sc_embedding_gather(table, flat_idx):
    n = flat_idx.shape[0]
    d = table.shape[1]
    dtype = table.dtype
    info = plsc.get_sparse_core_info()
    nw = info.num_cores * info.num_subcores
    sub = 128
    subs_per_chunk = 8
    chunk = sub * subs_per_chunk
    per_w = n // nw
    n_chunks = per_w // chunk
    idx2d = flat_idx.reshape(n // sub, sub)
    mesh = plsc.VectorSubcoreMesh(core_axis_name="c", subcore_axis_name="s")

    @functools.partial(
        pl.kernel,
        mesh=mesh,
        compiler_params=pltpu.CompilerParams(use_tc_tiling_on_sc=False),
        out_type=jax.ShapeDtypeStruct((n, d), dtype),
        scratch_types=[
            pltpu.VMEM((subs_per_chunk, sub), jnp.int32),
            pltpu.VMEM((chunk, d), dtype),
            pltpu.SemaphoreType.DMA,
        ],
    )
    def gather_kernel(table_hbm, idx_hbm, out_hbm, idx_v, rows_v, sem):
        wid = lax.axis_index("s") * info.num_cores + lax.axis_index("c")
        row_base = wid * per_w
        irow_base = row_base // sub

        def body(i, carry):
            row0 = pl.multiple_of(row_base + i * chunk, chunk)
            irow0 = pl.multiple_of(irow_base + i * subs_per_chunk,
                                   subs_per_chunk)
            pltpu.sync_copy(idx_hbm.at[pl.ds(irow0, subs_per_chunk)], idx_v)
            copies = [
                pltpu.async_copy(
                    table_hbm.at[idx_v.at[j]],
                    rows_v.at[pl.ds(j * sub, sub)],
                    sem,
                )
                for j in range(subs_per_chunk)
            ]
            for c in copies:
                c.wait()
            pltpu.sync_copy(rows_v, out_hbm.at[pl.ds(row0, chunk)])
            return carry

        lax.fori_loop(0, n_chunks, body, 0)

    return gather_kernel(table, idx2d)


def _gru_scan(x_lbd, ids_lb1, kern, rk, bias, reverse):
    L, B, D = x_lbd.shape

    def body(x_ref, ids_ref, k_ref, rk_ref, b_ref, out_ref, h_ref):
        i = pl.program_id(0)
        t = (L - 1 - i) if reverse else i

        @pl.when(i == 0)
        def _init():
            h_ref[...] = jnp.zeros_like(h_ref)

        h = h_ref[...]
        x_t = x_ref[0]
        gx = (
            jnp.dot(x_t, k_ref[...], preferred_element_type=jnp.float32)
            + b_ref[0:1, :]
        )
        gh = (
            jnp.dot(h.astype(jnp.bfloat16), rk_ref[...],
                    preferred_element_type=jnp.float32)
            + b_ref[1:2, :]
        )
        xz = gx[:, :H]
        xr = gx[:, H:2 * H]
        xh = gx[:, 2 * H:]
        hz = gh[:, :H]
        hr = gh[:, H:2 * H]
        hh = gh[:, 2 * H:]
        z = jax.nn.sigmoid(xz + hz)
        r = jax.nn.sigmoid(xr + hr)
        c = jnp.tanh(xh + r * hh)
        h_new = z * h + (1.0 - z) * c
        m = ids_ref[0] != 0
        h_new = jnp.where(m, h_new, h)
        h_ref[...] = h_new
        out_ref[:, pl.ds(t % 8, 1), :] = h_new[:, None, :]

    if reverse:
        in_idx = lambda i: (L - 1 - i, 0, 0)
        out_idx = lambda i: (0, (L - 1 - i) // 8, 0)
    else:
        in_idx = lambda i: (i, 0, 0)
        out_idx = lambda i: (0, i // 8, 0)

    return pl.pallas_call(
        body,
        grid=(L,),
        in_specs=[
            pl.BlockSpec((1, B, D), in_idx),
            pl.BlockSpec((1, B, 1), in_idx),
            pl.BlockSpec((D, 3 * H), lambda t: (0, 0)),
            pl.BlockSpec((H, 3 * H), lambda t: (0, 0)),
            pl.BlockSpec((2, 3 * H), lambda t: (0, 0)),
        ],
        out_specs=pl.BlockSpec((B, 8, H), out_idx),
        out_shape=jax.ShapeDtypeStruct((B, L, H), jnp.float32),
        scratch_shapes=[pltpu.VMEM((B, H), jnp.float32)],
    )(x_lbd, ids_lb1, kern, rk, bias)


def _heads(fwd, bwd, w, bb):
    B, L, _ = fwd.shape
    Bb = 64

    def body(f_ref, b_ref, w_ref, bias_ref, xg_ref, p_ref):
        xg = jnp.concatenate([f_ref[...], b_ref[...]], axis=-1)
        xg_ref[...] = xg
        flat = xg.reshape(Bb * L, 2 * H)
        logits = (
            jnp.dot(flat, w_ref[...], preferred_element_type=jnp.float32)
            + bias_ref[0:1, :]
        )
        p_ref[...] = jax.nn.sigmoid(logits)

    return pl.pallas_call(
        body,
        grid=(B // Bb,),
        in_specs=[
            pl.BlockSpec((Bb, L, H), lambda i: (i, 0, 0)),
            pl.BlockSpec((Bb, L, H), lambda i: (i, 0, 0)),
            pl.BlockSpec((2 * H, 2), lambda i: (0, 0)),
            pl.BlockSpec((1, 2), lambda i: (0, 0)),
        ],
        out_specs=[
            pl.BlockSpec((Bb, L, 2 * H), lambda i: (i, 0, 0)),
            pl.BlockSpec((Bb * L, 2), lambda i: (i, 0)),
        ],
        out_shape=[
            jax.ShapeDtypeStruct((B, L, 2 * H), jnp.float32),
            jax.ShapeDtypeStruct((B * L, 2), jnp.float32),
        ],
    )(fwd, bwd, w, bb)


def kernel(inputs, emb, k_f, rk_f, b_f, k_b, rk_b, b_b, W1, b1, W2, b2):
    B, L = inputs.shape
    ids = inputs.astype(jnp.int32)
    ids_t = ids.T
    x_flat = _sc_embedding_gather(emb.astype(jnp.bfloat16), ids_t.reshape(-1))
    x_lbd = x_flat.reshape(L, B, H)
    ids_lb1 = ids_t.reshape(L, B, 1)
    bf = jnp.bfloat16
    fwd = _gru_scan(x_lbd, ids_lb1, k_f.astype(bf), rk_f.astype(bf), b_f,
                    reverse=False)
    bwd = _gru_scan(x_lbd, ids_lb1, k_b.astype(bf), rk_b.astype(bf), b_b,
                    reverse=True)
    w = jnp.concatenate([W1, W2], axis=1)
    bb = jnp.concatenate([b1, b2]).reshape(1, 2)
    xgru, probs = _heads(fwd, bwd, w, bb)
    x1 = probs[:, 0].reshape(B, L, 1)
    x2 = probs[:, 1].reshape(B, L, 1)
    return (x1, x2, xgru)
